# Initial kernel scaffold; baseline (speedup 1.0000x reference)
import functools

import jax
import jax.numpy as jnp
from jax import lax
from jax.experimental import pallas as pl
from jax.experimental.pallas import tpu as pltpu
from jax.experimental.pallas import tpu_sc as plsc

NUM_CORES = 2
NUM_SUBCORES = 16
NUM_WORKERS = NUM_CORES * NUM_SUBCORES
CHUNK = 1024


def _make_gather(total_rows: int, row_dim: int):
    assert total_rows % (8 * NUM_WORKERS) == 0
    rows_per_worker = total_rows // NUM_WORKERS
    assert rows_per_worker % CHUNK == 0
    mesh = plsc.VectorSubcoreMesh(core_axis_name="c", subcore_axis_name="s")

    @functools.partial(
        pl.kernel,
        mesh=mesh,
        out_type=jax.ShapeDtypeStruct((total_rows, row_dim), jnp.float32),
        scratch_types=[
            pltpu.VMEM((CHUNK,), jnp.int32),
            pltpu.VMEM((CHUNK, row_dim), jnp.float32),
            pltpu.SemaphoreType.DMA,
        ],
    )
    def gather_kernel(table_hbm, idx_hbm, out_hbm, idx_v, rows_v, sem):
        wid = lax.axis_index("s") * NUM_CORES + lax.axis_index("c")
        base = wid * rows_per_worker

        @pl.loop(0, rows_per_worker, step=CHUNK)
        def _(c):
            start = base + c
            pltpu.sync_copy(idx_hbm.at[pl.ds(start, CHUNK)], idx_v)
            pltpu.async_copy(table_hbm.at[idx_v], rows_v, sem).wait()
            pltpu.sync_copy(rows_v, out_hbm.at[pl.ds(start, CHUNK)])

    return gather_kernel


def kernel(codes, tables):
    B, Q, T = codes.shape
    V = tables.shape[1]
    d = tables.shape[2]
    offs = (jnp.arange(Q, dtype=jnp.int32) * V)[None, :, None]
    idx = (codes + offs).transpose(0, 2, 1).reshape(B * T * Q)
    tab = tables.reshape(Q * V, d)
    rows = _make_gather(B * T * Q, d)(tab, idx)
    return rows.reshape(B, T, Q * d)


# speedup vs baseline: 1.5144x; 1.5144x over previous
import functools

import jax
import jax.numpy as jnp
from jax import lax
from jax.experimental import pallas as pl
from jax.experimental.pallas import tpu as pltpu
from jax.experimental.pallas import tpu_sc as plsc

NUM_CORES = 2
NUM_SUBCORES = 16
NUM_WORKERS = NUM_CORES * NUM_SUBCORES
CHUNK = 1024


def _make_gather(total_rows: int, row_dim: int):
    assert total_rows % (8 * NUM_WORKERS) == 0
    rows_per_worker = total_rows // NUM_WORKERS
    assert rows_per_worker % CHUNK == 0
    mesh = plsc.VectorSubcoreMesh(core_axis_name="c", subcore_axis_name="s")

    @functools.partial(
        pl.kernel,
        mesh=mesh,
        out_type=jax.ShapeDtypeStruct((total_rows, row_dim), jnp.float32),
        scratch_types=[
            pltpu.VMEM((CHUNK,), jnp.int32),
            pltpu.VMEM((CHUNK, row_dim), jnp.float32),
            pltpu.SemaphoreType.DMA,
        ],
        compiler_params=pltpu.CompilerParams(use_tc_tiling_on_sc=False),
    )
    def gather_kernel(table_hbm, idx_hbm, out_hbm, idx_v, rows_v, sem):
        wid = lax.axis_index("s") * NUM_CORES + lax.axis_index("c")
        base = wid * rows_per_worker

        @pl.loop(0, rows_per_worker, step=CHUNK)
        def _(c):
            start = base + c
            pltpu.sync_copy(idx_hbm.at[pl.ds(start, CHUNK)], idx_v)
            pltpu.async_copy(table_hbm.at[idx_v], rows_v, sem).wait()
            pltpu.sync_copy(rows_v, out_hbm.at[pl.ds(start, CHUNK)])

    return gather_kernel


def kernel(codes, tables):
    B, Q, T = codes.shape
    V = tables.shape[1]
    d = tables.shape[2]
    offs = (jnp.arange(Q, dtype=jnp.int32) * V)[None, :, None]
    idx = (codes + offs).transpose(0, 2, 1).reshape(B * T * Q)
    tab = tables.reshape(Q * V, d)
    rows = _make_gather(B * T * Q, d)(tab, idx)
    return rows.reshape(B, T, Q * d)


# speedup vs baseline: 1.5561x; 1.0276x over previous
import functools

import jax
import jax.numpy as jnp
from jax import lax
from jax.experimental import pallas as pl
from jax.experimental.pallas import tpu as pltpu
from jax.experimental.pallas import tpu_sc as plsc

NUM_CORES = 2
NUM_SUBCORES = 16
NUM_WORKERS = NUM_CORES * NUM_SUBCORES
CHUNK = 1024
NBUF = 4


def _make_gather(total_rows: int, row_dim: int):
    assert total_rows % (8 * NUM_WORKERS) == 0
    rows_per_worker = total_rows // NUM_WORKERS
    assert rows_per_worker % CHUNK == 0
    nchunks = rows_per_worker // CHUNK
    mesh = plsc.VectorSubcoreMesh(core_axis_name="c", subcore_axis_name="s")

    @functools.partial(
        pl.kernel,
        mesh=mesh,
        out_type=jax.ShapeDtypeStruct((total_rows, row_dim), jnp.float32),
        scratch_types=(
            [pltpu.VMEM((rows_per_worker,), jnp.int32)]
            + [pltpu.VMEM((CHUNK, row_dim), jnp.float32)] * NBUF
            + [pltpu.SemaphoreType.DMA] * (2 * NBUF)
        ),
        compiler_params=pltpu.CompilerParams(use_tc_tiling_on_sc=False),
    )
    def gather_kernel(table_hbm, idx_hbm, out_hbm, idx_v, *bufs_and_sems):
        rows = bufs_and_sems[:NBUF]
        gsems = bufs_and_sems[NBUF : 2 * NBUF]
        ssems = bufs_and_sems[2 * NBUF :]
        wid = lax.axis_index("s") * NUM_CORES + lax.axis_index("c")
        base = wid * rows_per_worker

        pltpu.sync_copy(idx_hbm.at[pl.ds(base, rows_per_worker)], idx_v)

        gathers = [None] * nchunks
        stores = [None] * nchunks

        def start_gather(c):
            b = c % NBUF
            gathers[c] = pltpu.async_copy(
                table_hbm.at[idx_v.at[pl.ds(c * CHUNK, CHUNK)]],
                rows[b],
                gsems[b],
            )

        for c in range(min(NBUF, nchunks)):
            start_gather(c)
        for c in range(nchunks):
            b = c % NBUF
            gathers[c].wait()
            stores[c] = pltpu.async_copy(
                rows[b], out_hbm.at[pl.ds(base + c * CHUNK, CHUNK)], ssems[b]
            )
            if c + NBUF < nchunks:
                stores[c].wait()
                start_gather(c + NBUF)
        for c in range(max(0, nchunks - NBUF), nchunks):
            stores[c].wait()

    return gather_kernel


def kernel(codes, tables):
    B, Q, T = codes.shape
    V = tables.shape[1]
    d = tables.shape[2]
    offs = (jnp.arange(Q, dtype=jnp.int32) * V)[None, :, None]
    idx = (codes + offs).transpose(0, 2, 1).reshape(B * T * Q)
    tab = tables.reshape(Q * V, d)
    rows = _make_gather(B * T * Q, d)(tab, idx)
    return rows.reshape(B, T, Q * d)


# speedup vs baseline: 1.5658x; 1.0062x over previous
import functools

import jax
import jax.numpy as jnp
from jax import lax
from jax.experimental import pallas as pl
from jax.experimental.pallas import tpu as pltpu
from jax.experimental.pallas import tpu_sc as plsc

NUM_CORES = 2
NUM_SUBCORES = 16
NUM_WORKERS = NUM_CORES * NUM_SUBCORES
CHUNK = 1024
NBUF = 4


def _make_lookup(B: int, Q: int, T: int, V: int, dq: int):
    pairs = B * Q
    assert pairs % NUM_WORKERS == 0
    pairs_per_worker = pairs // NUM_WORKERS
    assert T % CHUNK == 0
    chunks_per_pair = T // CHUNK
    nsteps = pairs_per_worker * chunks_per_pair
    idx_per_worker = pairs_per_worker * T
    mesh = plsc.VectorSubcoreMesh(core_axis_name="c", subcore_axis_name="s")

    @functools.partial(
        pl.kernel,
        mesh=mesh,
        out_type=jax.ShapeDtypeStruct((B * T, Q * dq), jnp.float32),
        scratch_types=(
            [pltpu.VMEM((idx_per_worker,), jnp.int32)]
            + [pltpu.VMEM((CHUNK, dq), jnp.float32)] * NBUF
            + [pltpu.SemaphoreType.DMA] * (2 * NBUF)
        ),
        compiler_params=pltpu.CompilerParams(use_tc_tiling_on_sc=False),
    )
    def lookup_kernel(tables_hbm, codes_hbm, out_hbm, idx_v, *bufs_and_sems):
        rows = bufs_and_sems[:NBUF]
        gsems = bufs_and_sems[NBUF : 2 * NBUF]
        ssems = bufs_and_sems[2 * NBUF :]
        wid = lax.axis_index("s") * NUM_CORES + lax.axis_index("c")
        pair0 = wid * pairs_per_worker

        pltpu.sync_copy(codes_hbm.at[pl.ds(pair0 * T, idx_per_worker)], idx_v)

        def step_coords(j):
            pair = pair0 + j // chunks_per_pair
            t0 = (j % chunks_per_pair) * CHUNK
            b = pair // Q
            q = pair % Q
            return b, q, t0

        gathers = [None] * nsteps
        stores = [None] * nsteps

        def start_gather(j):
            _, q, _ = step_coords(j)
            bb = j % NBUF
            gathers[j] = pltpu.async_copy(
                tables_hbm.at[q].at[idx_v.at[pl.ds(j * CHUNK, CHUNK)]],
                rows[bb],
                gsems[bb],
            )

        for j in range(min(NBUF, nsteps)):
            start_gather(j)
        for j in range(nsteps):
            b, q, t0 = step_coords(j)
            bb = j % NBUF
            gathers[j].wait()
            stores[j] = pltpu.async_copy(
                rows[bb],
                out_hbm.at[pl.ds(b * T + t0, CHUNK), pl.ds(q * dq, dq)],
                ssems[bb],
            )
            if j + NBUF < nsteps:
                stores[j].wait()
                start_gather(j + NBUF)
        for j in range(max(0, nsteps - NBUF), nsteps):
            stores[j].wait()

    return lookup_kernel


def kernel(codes, tables):
    B, Q, T = codes.shape
    V = tables.shape[1]
    dq = tables.shape[2]
    out = _make_lookup(B, Q, T, V, dq)(tables, codes.reshape(B * Q * T))
    return out.reshape(B, T, Q * dq)
